# Initial kernel scaffold; baseline (speedup 1.0000x reference)
import jax
import jax.numpy as jnp
from jax.experimental import pallas as pl


def kernel(V, E, edge_index, A1_w, A1_b, A2_w, A2_b, Bm_w, Bm_b, Cm_w, Cm_b, Pa_w, Pa_b, Pb_w, Pb_b):
    raise NotImplementedError("write your pallas kernel here")


# speedup vs baseline: 2.5349x; 2.5349x over previous
import jax
import jax.numpy as jnp
from jax import lax
from jax.experimental import pallas as pl
from jax.experimental.pallas import tpu as pltpu
from jax.experimental.pallas import tpu_sc as plsc

N_NODES = 10000
N_EDGES = 320000
D_NODE = 128
D_EDGE = 16
EPS = 1e-5

NC = 2
NS = 16
NW = NC * NS
EPT = N_EDGES // NW
EB = 80
NBLK = EPT // EB
N_PAD = 10240
ROWS_PT = N_PAD // NS


def _pool_body(v_ref, paw_ref, pab_ref, pbw_ref, pbb_ref, out_ref):
    v = v_ref[...]
    v = jnp.where(v >= 0.0, v, 0.2 * v)
    h = jnp.dot(v, paw_ref[...], preferred_element_type=jnp.float32) + pab_ref[...]
    h = jnp.where(h >= 0.0, h, 0.2 * h)
    out_ref[...] = jnp.dot(h, pbw_ref[...], preferred_element_type=jnp.float32) + pbb_ref[...]


def _pool(V, Pa_w, Pa_b, Pb_w, Pb_b):
    blk = 2000
    grid = N_NODES // blk
    wspec = pl.BlockSpec((D_NODE, D_NODE), lambda i: (0, 0))
    bspec = pl.BlockSpec((1, D_NODE), lambda i: (0, 0))
    return pl.pallas_call(
        _pool_body,
        grid=(grid,),
        in_specs=[pl.BlockSpec((blk, D_NODE), lambda i: (i, 0)),
                  wspec, bspec, wspec, bspec],
        out_specs=pl.BlockSpec((blk, D_NODE), lambda i: (i, 0)),
        out_shape=jax.ShapeDtypeStruct((N_NODES, D_NODE), jnp.float32),
    )(V, Pa_w, Pa_b.reshape(1, D_NODE), Pb_w, Pb_b.reshape(1, D_NODE))


def _msg_body(e_ref, a1w, a1b, a2w, a2b, bmw, bmb, cmw, cmb, scale_ref, shift_ref):
    e = e_ref[...]
    x = jnp.maximum(jnp.dot(e, a1w[...], preferred_element_type=jnp.float32) + a1b[...], 0.0)
    x = jnp.dot(x, a2w[...], preferred_element_type=jnp.float32) + a2b[...]
    scale_ref[...] = jax.nn.sigmoid(
        jnp.dot(x, bmw[...], preferred_element_type=jnp.float32) + bmb[...])
    shift_ref[...] = jnp.dot(x, cmw[...], preferred_element_type=jnp.float32) + cmb[...]


def _edge_messages(E, A1_w, A1_b, A2_w, A2_b, Bm_w, Bm_b, Cm_w, Cm_b):
    blk = 8000
    grid = N_EDGES // blk
    wee = pl.BlockSpec((D_EDGE, D_EDGE), lambda i: (0, 0))
    bee = pl.BlockSpec((1, D_EDGE), lambda i: (0, 0))
    wen = pl.BlockSpec((D_EDGE, D_NODE), lambda i: (0, 0))
    ben = pl.BlockSpec((1, D_NODE), lambda i: (0, 0))
    ospec = pl.BlockSpec((blk, D_NODE), lambda i: (i, 0))
    return pl.pallas_call(
        _msg_body,
        grid=(grid,),
        in_specs=[pl.BlockSpec((blk, D_EDGE), lambda i: (i, 0)),
                  wee, bee, wee, bee, wen, ben, wen, ben],
        out_specs=[ospec, ospec],
        out_shape=[jax.ShapeDtypeStruct((N_EDGES, D_NODE), jnp.float32),
                   jax.ShapeDtypeStruct((N_EDGES, D_NODE), jnp.float32)],
    )(E, A1_w, A1_b.reshape(1, D_EDGE), A2_w, A2_b.reshape(1, D_EDGE),
      Bm_w, Bm_b.reshape(1, D_NODE), Cm_w, Cm_b.reshape(1, D_NODE))


def _cnt_body(dst_hbm, z128_hbm, ones_hbm, cnt_out, dstv, onesv, cnts):
    c = lax.axis_index("c")
    s = lax.axis_index("s")
    wid = c * NS + s
    row0 = s * ROWS_PT

    pltpu.sync_copy(z128_hbm.at[pl.ds(row0, ROWS_PT)], cnts.at[pl.ds(row0, ROWS_PT)])
    pltpu.sync_copy(ones_hbm, onesv)

    plsc.subcore_barrier()

    base0 = wid * EPT

    @pl.loop(0, NBLK)
    def _(i):
        pltpu.sync_copy(dst_hbm.at[pl.ds(base0 + i * EB, EB)], dstv)
        pltpu.sync_copy(onesv, cnts.at[dstv], add=True)

    plsc.subcore_barrier()

    pltpu.sync_copy(cnts.at[pl.ds(row0, ROWS_PT)],
                    cnt_out.at[pl.ds(c * N_PAD + row0, ROWS_PT)])


def _sc_counts(dst, z128, ones128):
    mesh = plsc.VectorSubcoreMesh(core_axis_name="c", subcore_axis_name="s")
    fn = pl.kernel(
        _cnt_body,
        mesh=mesh,
        out_type=jax.ShapeDtypeStruct((NC * N_PAD, D_NODE), jnp.float32),
        scratch_types=[
            pltpu.VMEM((EB,), jnp.int32),
            pltpu.VMEM((EB, D_NODE), jnp.float32),
            pltpu.VMEM_SHARED((N_PAD, D_NODE), jnp.float32),
        ],
    )
    return fn(dst, z128, ones128)


def _sc_body(vp_hbm, scale_hbm, shift_hbm, src_hbm, dst_hbm, z128_hbm,
             acc_out,
             srcv, dstv, rows, scv, shv, accs, gsem, ssem, hsem):
    c = lax.axis_index("c")
    s = lax.axis_index("s")
    wid = c * NS + s
    row0 = s * ROWS_PT

    pltpu.sync_copy(z128_hbm.at[pl.ds(row0, ROWS_PT)], accs.at[pl.ds(row0, ROWS_PT)])

    plsc.subcore_barrier()

    base0 = wid * EPT

    @pl.loop(0, NBLK)
    def _(i):
        base = base0 + i * EB
        pltpu.sync_copy(src_hbm.at[pl.ds(base, EB)], srcv)
        pltpu.sync_copy(dst_hbm.at[pl.ds(base, EB)], dstv)
        g = pltpu.async_copy(vp_hbm.at[srcv], rows, gsem)
        c1 = pltpu.async_copy(scale_hbm.at[pl.ds(base, EB)], scv, ssem)
        c2 = pltpu.async_copy(shift_hbm.at[pl.ds(base, EB)], shv, hsem)
        g.wait()
        c1.wait()
        c2.wait()

        @pl.loop(0, EB)
        def _(r):
            for ch in range(D_NODE // 16):
                slc = (pl.ds(r, 1), pl.ds(ch * 16, 16))
                v = rows.at[slc[0], slc[1]][...]
                sc_ = scv.at[slc[0], slc[1]][...]
                sh_ = shv.at[slc[0], slc[1]][...]
                m = jnp.maximum(sc_ * v + sh_, EPS)
                rows.at[slc[0], slc[1]][...] = m * m

        pltpu.sync_copy(rows, accs.at[dstv], add=True)

    plsc.subcore_barrier()

    pltpu.sync_copy(accs.at[pl.ds(row0, ROWS_PT)],
                    acc_out.at[pl.ds(c * N_PAD + row0, ROWS_PT)])


def _sc_aggregate(Vp, scale, shift, src, dst, z128):
    mesh = plsc.VectorSubcoreMesh(core_axis_name="c", subcore_axis_name="s")
    fn = pl.kernel(
        _sc_body,
        mesh=mesh,
        out_type=jax.ShapeDtypeStruct((NC * N_PAD, D_NODE), jnp.float32),
        scratch_types=[
            pltpu.VMEM((EB,), jnp.int32),
            pltpu.VMEM((EB,), jnp.int32),
            pltpu.VMEM((EB, D_NODE), jnp.float32),
            pltpu.VMEM((EB, D_NODE), jnp.float32),
            pltpu.VMEM((EB, D_NODE), jnp.float32),
            pltpu.VMEM_SHARED((N_PAD, D_NODE), jnp.float32),
            pltpu.SemaphoreType.DMA,
            pltpu.SemaphoreType.DMA,
            pltpu.SemaphoreType.DMA,
        ],
    )
    return fn(Vp, scale, shift, src, dst, z128)


def _fin_body(a0_ref, a1_ref, c0_ref, c1_ref, out_ref):
    a = a0_ref[...] + a1_ref[...]
    cnt = c0_ref[...][:, 0:1] + c1_ref[...][:, 0:1]
    mean = a / jnp.maximum(cnt, 1.0)
    out_ref[...] = jnp.sqrt(jnp.maximum(mean, 1e-12))


def _finalize(acc_flat, cnt_flat):
    blk = 2048
    grid = N_PAD // blk
    return pl.pallas_call(
        _fin_body,
        grid=(grid,),
        in_specs=[pl.BlockSpec((blk, D_NODE), lambda i: (i, 0)),
                  pl.BlockSpec((blk, D_NODE), lambda i: (i + grid, 0)),
                  pl.BlockSpec((blk, D_NODE), lambda i: (i, 0)),
                  pl.BlockSpec((blk, D_NODE), lambda i: (i + grid, 0))],
        out_specs=pl.BlockSpec((blk, D_NODE), lambda i: (i, 0)),
        out_shape=jax.ShapeDtypeStruct((N_PAD, D_NODE), jnp.float32),
    )(acc_flat, acc_flat, cnt_flat, cnt_flat)


def kernel(V, E, edge_index, A1_w, A1_b, A2_w, A2_b, Bm_w, Bm_b, Cm_w, Cm_b,
           Pa_w, Pa_b, Pb_w, Pb_b):
    Vp = _pool(V, Pa_w, Pa_b, Pb_w, Pb_b)
    scale, shift = _edge_messages(E, A1_w, A1_b, A2_w, A2_b, Bm_w, Bm_b, Cm_w, Cm_b)
    src = edge_index[0]
    dst = edge_index[1]
    z128 = jnp.zeros((N_PAD, D_NODE), jnp.float32)
    ones128 = jnp.ones((EB, D_NODE), jnp.float32)
    cnt_flat = _sc_counts(dst, z128, ones128)
    acc_flat = _sc_aggregate(Vp, scale, shift, src, dst, z128)
    return _finalize(acc_flat, cnt_flat)[:N_NODES]


# speedup vs baseline: 2.8485x; 1.1237x over previous
import jax
import jax.numpy as jnp
from jax import lax
from jax.experimental import pallas as pl
from jax.experimental.pallas import tpu as pltpu
from jax.experimental.pallas import tpu_sc as plsc

N_NODES = 10000
N_EDGES = 320000
D_NODE = 128
D_EDGE = 16
EPS = 1e-5

NC = 2
NS = 16
NW = NC * NS
EPT = N_EDGES // NW
EB = 40
NBLK = EPT // EB
EBC = 80
NBLKC = EPT // EBC
N_PAD = 10240
ROWS_PT = N_PAD // NS


def _pool_body(v_ref, paw_ref, pab_ref, pbw_ref, pbb_ref, out_ref):
    v = v_ref[...]
    v = jnp.where(v >= 0.0, v, 0.2 * v)
    h = jnp.dot(v, paw_ref[...], preferred_element_type=jnp.float32) + pab_ref[...]
    h = jnp.where(h >= 0.0, h, 0.2 * h)
    out_ref[...] = jnp.dot(h, pbw_ref[...], preferred_element_type=jnp.float32) + pbb_ref[...]


def _pool(V, Pa_w, Pa_b, Pb_w, Pb_b):
    blk = 2000
    grid = N_NODES // blk
    wspec = pl.BlockSpec((D_NODE, D_NODE), lambda i: (0, 0))
    bspec = pl.BlockSpec((1, D_NODE), lambda i: (0, 0))
    return pl.pallas_call(
        _pool_body,
        grid=(grid,),
        in_specs=[pl.BlockSpec((blk, D_NODE), lambda i: (i, 0)),
                  wspec, bspec, wspec, bspec],
        out_specs=pl.BlockSpec((blk, D_NODE), lambda i: (i, 0)),
        out_shape=jax.ShapeDtypeStruct((N_NODES, D_NODE), jnp.float32),
    )(V, Pa_w, Pa_b.reshape(1, D_NODE), Pb_w, Pb_b.reshape(1, D_NODE))


def _msg_body(e_ref, a1w, a1b, a2w, a2b, bmw, bmb, cmw, cmb, scale_ref, shift_ref):
    e = e_ref[...]
    x = jnp.maximum(jnp.dot(e, a1w[...], preferred_element_type=jnp.float32) + a1b[...], 0.0)
    x = jnp.dot(x, a2w[...], preferred_element_type=jnp.float32) + a2b[...]
    scale_ref[...] = jax.nn.sigmoid(
        jnp.dot(x, bmw[...], preferred_element_type=jnp.float32) + bmb[...])
    shift_ref[...] = jnp.dot(x, cmw[...], preferred_element_type=jnp.float32) + cmb[...]


def _edge_messages(E, A1_w, A1_b, A2_w, A2_b, Bm_w, Bm_b, Cm_w, Cm_b):
    blk = 8000
    grid = N_EDGES // blk
    wee = pl.BlockSpec((D_EDGE, D_EDGE), lambda i: (0, 0))
    bee = pl.BlockSpec((1, D_EDGE), lambda i: (0, 0))
    wen = pl.BlockSpec((D_EDGE, D_NODE), lambda i: (0, 0))
    ben = pl.BlockSpec((1, D_NODE), lambda i: (0, 0))
    ospec = pl.BlockSpec((blk, D_NODE), lambda i: (i, 0))
    return pl.pallas_call(
        _msg_body,
        grid=(grid,),
        in_specs=[pl.BlockSpec((blk, D_EDGE), lambda i: (i, 0)),
                  wee, bee, wee, bee, wen, ben, wen, ben],
        out_specs=[ospec, ospec],
        out_shape=[jax.ShapeDtypeStruct((N_EDGES, D_NODE), jnp.float32),
                   jax.ShapeDtypeStruct((N_EDGES, D_NODE), jnp.float32)],
    )(E, A1_w, A1_b.reshape(1, D_EDGE), A2_w, A2_b.reshape(1, D_EDGE),
      Bm_w, Bm_b.reshape(1, D_NODE), Cm_w, Cm_b.reshape(1, D_NODE))


def _cnt_body(dst_hbm, z128_hbm, ones_hbm, cnt_out, dstv, onesv, cnts):
    c = lax.axis_index("c")
    s = lax.axis_index("s")
    wid = c * NS + s
    row0 = s * ROWS_PT

    pltpu.sync_copy(z128_hbm.at[pl.ds(row0, ROWS_PT)], cnts.at[pl.ds(row0, ROWS_PT)])
    pltpu.sync_copy(ones_hbm, onesv)

    plsc.subcore_barrier()

    base0 = wid * EPT

    @pl.loop(0, NBLKC)
    def _(i):
        pltpu.sync_copy(dst_hbm.at[pl.ds(base0 + i * EBC, EBC)], dstv)
        pltpu.sync_copy(onesv, cnts.at[dstv], add=True)

    plsc.subcore_barrier()

    pltpu.sync_copy(cnts.at[pl.ds(row0, ROWS_PT)],
                    cnt_out.at[pl.ds(c * N_PAD + row0, ROWS_PT)])


def _sc_counts(dst, z128, ones128):
    mesh = plsc.VectorSubcoreMesh(core_axis_name="c", subcore_axis_name="s")
    fn = pl.kernel(
        _cnt_body,
        mesh=mesh,
        out_type=jax.ShapeDtypeStruct((NC * N_PAD, D_NODE), jnp.float32),
        scratch_types=[
            pltpu.VMEM((EBC,), jnp.int32),
            pltpu.VMEM((EBC, D_NODE), jnp.float32),
            pltpu.VMEM_SHARED((N_PAD, D_NODE), jnp.float32),
        ],
    )
    return fn(dst, z128, ones128)


def _sc_body(vp_hbm, scale_hbm, shift_hbm, src_hbm, dst_hbm, z128_hbm,
             acc_out,
             srcv0, dstv0, rows0, scv0, shv0,
             srcv1, dstv1, rows1, scv1, shv1,
             accs, gsem0, ssem0, hsem0, gsem1, ssem1, hsem1):
    c = lax.axis_index("c")
    s = lax.axis_index("s")
    wid = c * NS + s
    row0 = s * ROWS_PT

    pltpu.sync_copy(z128_hbm.at[pl.ds(row0, ROWS_PT)], accs.at[pl.ds(row0, ROWS_PT)])

    plsc.subcore_barrier()

    base0 = wid * EPT

    def load(i, srcv, dstv, rows, scv, shv, gsem, ssem, hsem):
        base = base0 + i * EB
        pltpu.sync_copy(src_hbm.at[pl.ds(base, EB)], srcv)
        pltpu.sync_copy(dst_hbm.at[pl.ds(base, EB)], dstv)
        pltpu.async_copy(vp_hbm.at[srcv], rows, gsem)
        pltpu.async_copy(scale_hbm.at[pl.ds(base, EB)], scv, ssem)
        pltpu.async_copy(shift_hbm.at[pl.ds(base, EB)], shv, hsem)

    def process(srcv, dstv, rows, scv, shv, gsem, ssem, hsem):
        pltpu.make_async_copy(vp_hbm.at[srcv], rows, gsem).wait()
        pltpu.make_async_copy(scale_hbm.at[pl.ds(0, EB)], scv, ssem).wait()
        pltpu.make_async_copy(shift_hbm.at[pl.ds(0, EB)], shv, hsem).wait()

        @pl.loop(0, EB)
        def _(r):
            for ch in range(D_NODE // 16):
                slc = (pl.ds(r, 1), pl.ds(ch * 16, 16))
                v = rows.at[slc[0], slc[1]][...]
                sc_ = scv.at[slc[0], slc[1]][...]
                sh_ = shv.at[slc[0], slc[1]][...]
                m = jnp.maximum(sc_ * v + sh_, EPS)
                rows.at[slc[0], slc[1]][...] = m * m

        pltpu.sync_copy(rows, accs.at[dstv], add=True)

    buf0 = (srcv0, dstv0, rows0, scv0, shv0, gsem0, ssem0, hsem0)
    buf1 = (srcv1, dstv1, rows1, scv1, shv1, gsem1, ssem1, hsem1)

    load(0, *buf0)

    @pl.loop(0, NBLK // 2 - 1)
    def _(j):
        load(2 * j + 1, *buf1)
        process(*buf0)
        load(2 * j + 2, *buf0)
        process(*buf1)

    load(NBLK - 1, *buf1)
    process(*buf0)
    process(*buf1)

    plsc.subcore_barrier()

    pltpu.sync_copy(accs.at[pl.ds(row0, ROWS_PT)],
                    acc_out.at[pl.ds(c * N_PAD + row0, ROWS_PT)])


def _sc_aggregate(Vp, scale, shift, src, dst, z128):
    mesh = plsc.VectorSubcoreMesh(core_axis_name="c", subcore_axis_name="s")
    fn = pl.kernel(
        _sc_body,
        mesh=mesh,
        out_type=jax.ShapeDtypeStruct((NC * N_PAD, D_NODE), jnp.float32),
        scratch_types=[
            pltpu.VMEM((EB,), jnp.int32),
            pltpu.VMEM((EB,), jnp.int32),
            pltpu.VMEM((EB, D_NODE), jnp.float32),
            pltpu.VMEM((EB, D_NODE), jnp.float32),
            pltpu.VMEM((EB, D_NODE), jnp.float32),
            pltpu.VMEM((EB,), jnp.int32),
            pltpu.VMEM((EB,), jnp.int32),
            pltpu.VMEM((EB, D_NODE), jnp.float32),
            pltpu.VMEM((EB, D_NODE), jnp.float32),
            pltpu.VMEM((EB, D_NODE), jnp.float32),
            pltpu.VMEM_SHARED((N_PAD, D_NODE), jnp.float32),
            pltpu.SemaphoreType.DMA,
            pltpu.SemaphoreType.DMA,
            pltpu.SemaphoreType.DMA,
            pltpu.SemaphoreType.DMA,
            pltpu.SemaphoreType.DMA,
            pltpu.SemaphoreType.DMA,
        ],
    )
    return fn(Vp, scale, shift, src, dst, z128)


def _fin_body(a0_ref, a1_ref, c0_ref, c1_ref, out_ref):
    a = a0_ref[...] + a1_ref[...]
    cnt = c0_ref[...][:, 0:1] + c1_ref[...][:, 0:1]
    mean = a / jnp.maximum(cnt, 1.0)
    out_ref[...] = jnp.sqrt(jnp.maximum(mean, 1e-12))


def _finalize(acc_flat, cnt_flat):
    blk = 2048
    grid = N_PAD // blk
    return pl.pallas_call(
        _fin_body,
        grid=(grid,),
        in_specs=[pl.BlockSpec((blk, D_NODE), lambda i: (i, 0)),
                  pl.BlockSpec((blk, D_NODE), lambda i: (i + grid, 0)),
                  pl.BlockSpec((blk, D_NODE), lambda i: (i, 0)),
                  pl.BlockSpec((blk, D_NODE), lambda i: (i + grid, 0))],
        out_specs=pl.BlockSpec((blk, D_NODE), lambda i: (i, 0)),
        out_shape=jax.ShapeDtypeStruct((N_PAD, D_NODE), jnp.float32),
    )(acc_flat, acc_flat, cnt_flat, cnt_flat)


def kernel(V, E, edge_index, A1_w, A1_b, A2_w, A2_b, Bm_w, Bm_b, Cm_w, Cm_b,
           Pa_w, Pa_b, Pb_w, Pb_b):
    Vp = _pool(V, Pa_w, Pa_b, Pb_w, Pb_b)
    scale, shift = _edge_messages(E, A1_w, A1_b, A2_w, A2_b, Bm_w, Bm_b, Cm_w, Cm_b)
    src = edge_index[0]
    dst = edge_index[1]
    z128 = jnp.zeros((N_PAD, D_NODE), jnp.float32)
    ones128 = jnp.ones((EBC, D_NODE), jnp.float32)
    cnt_flat = _sc_counts(dst, z128, ones128)
    acc_flat = _sc_aggregate(Vp, scale, shift, src, dst, z128)
    return _finalize(acc_flat, cnt_flat)[:N_NODES]
